# Initial kernel scaffold; baseline (speedup 1.0000x reference)
import jax
import jax.numpy as jnp
from jax.experimental import pallas as pl


def kernel(x, edge_index, W_fc1, b_fc1, W_c1, b_c1, W_c2, b_c2, W_c3, b_c3, W_fc2, b_fc2):
    raise NotImplementedError("write your pallas kernel here")


# speedup vs baseline: 1.9604x; 1.9604x over previous
import jax
import jax.numpy as jnp
from jax.experimental import pallas as pl


def _final_proj_kernel(h_ref, w_ref, b_ref, o_ref):
    o_ref[...] = h_ref[...] @ w_ref[...] + b_ref[...]


def kernel(x, edge_index, W_fc1, b_fc1, W_c1, b_c1, W_c2, b_c2, W_c3, b_c3, W_fc2, b_fc2):
    n = x.shape[0]
    src, dst = edge_index[0], edge_index[1]
    deg = jax.ops.segment_sum(jnp.ones_like(dst, jnp.float32), dst, num_segments=n) + 1.0
    dinv = jax.lax.rsqrt(deg)[:, None]

    def prop(u):
        v = dinv * u
        acc = jax.ops.segment_sum(v[src], dst, num_segments=n)
        return dinv * (acc + v)

    h = x @ W_fc1 + b_fc1
    for _ in range(4):
        h = jax.nn.relu(prop(h) @ W_c1 + b_c1)
        h = jax.nn.relu(prop(h @ W_c2) + b_c2)
        h = jax.nn.relu(prop(h @ W_c3) + b_c3)

    npad = 50048
    hp = jnp.zeros((npad, 32), h.dtype).at[:n].set(h)
    out = pl.pallas_call(
        _final_proj_kernel,
        out_shape=jax.ShapeDtypeStruct((npad, 1), jnp.float32),
        grid=(npad // 3128,),
        in_specs=[
            pl.BlockSpec((3128, 32), lambda i: (i, 0)),
            pl.BlockSpec((32, 1), lambda i: (0, 0)),
            pl.BlockSpec((1,), lambda i: (0,)),
        ],
        out_specs=pl.BlockSpec((3128, 1), lambda i: (i, 0)),
    )(hp, W_fc2, b_fc2)
    return out[:n]


# speedup vs baseline: 20.9005x; 10.6615x over previous
import functools

import jax
import jax.numpy as jnp
from jax import lax
from jax.experimental import pallas as pl
from jax.experimental.pallas import tpu as pltpu
from jax.experimental.pallas import tpu_sc as plsc

N = 50000
NPAD = 51200
TROWS = NPAD // 16
ZR = 40
E = 1600000
EROWS = 12544
EPAD = EROWS * 128
G = 8
PT_ROWS = EROWS // 16
DG_ROWS = EROWS // 32
DG_GROUPS = DG_ROWS // G
BN = 3200
GRID = NPAD // BN

_MESH = plsc.VectorSubcoreMesh(core_axis_name="c", subcore_axis_name="s")
_SC_PARAMS = pltpu.CompilerParams(use_tc_tiling_on_sc=False)


def _fill_zeros(ref, rows, cols):
    z = jnp.zeros((16,), jnp.float32)
    for r in range(rows):
        for k in range(cols // 16):
            ref[r, pl.ds(k * 16, 16)] = z


@functools.partial(
    pl.kernel,
    out_type=jax.ShapeDtypeStruct((2, NPAD), jnp.float32),
    mesh=_MESH,
    scratch_types=[
        pltpu.VMEM_SHARED((NPAD,), jnp.float32),
        pltpu.VMEM((G, 128), jnp.int32),
        pltpu.VMEM((128,), jnp.float32),
        pltpu.VMEM((TROWS,), jnp.float32),
        pltpu.SemaphoreType.DMA,
    ],
    compiler_params=_SC_PARAMS,
)
def _deg_pass(dstq, out, acc, dstb, ones, zbuf, ssem):
    c = lax.axis_index("c")
    s = lax.axis_index("s")
    one = jnp.ones((16,), jnp.float32)
    z = jnp.zeros((16,), jnp.float32)
    for i in range(8):
        ones[pl.ds(i * 16, 16)] = one
    for i in range(TROWS // 16):
        zbuf[pl.ds(i * 16, 16)] = z
    pltpu.sync_copy(zbuf, acc.at[pl.ds(s * TROWS, TROWS)])
    plsc.subcore_barrier()

    base = (s * 2 + c) * DG_ROWS

    def body(g, carry):
        r0 = base + g * G
        pltpu.sync_copy(dstq.at[pl.ds(r0, G)], dstb)
        cps = [pltpu.async_copy(ones, acc.at[dstb.at[j]], ssem, add=True)
               for j in range(G)]
        for cp in cps:
            cp.wait()
        return carry

    lax.fori_loop(0, DG_GROUPS, body, 0)
    plsc.subcore_barrier()
    pltpu.sync_copy(acc.at[pl.ds(s * TROWS, TROWS)],
                    out.at[c, pl.ds(s * TROWS, TROWS)])


def _make_prop(w2, pg):
    @functools.partial(
        pl.kernel,
        out_type=jax.ShapeDtypeStruct((2, NPAD, w2), jnp.float32),
        mesh=_MESH,
        scratch_types=[
            pltpu.VMEM_SHARED((NPAD, w2), jnp.float32),
            pltpu.VMEM((pg, 128), jnp.int32),
            pltpu.VMEM((pg, 128), jnp.int32),
            pltpu.VMEM((pg, 128, w2), jnp.float32),
            pltpu.VMEM((ZR, w2), jnp.float32),
            pltpu.SemaphoreType.DMA,
            pltpu.SemaphoreType.DMA,
        ],
        name=f"sc_prop{w2}",
        compiler_params=_SC_PARAMS,
    )
    def prop(vh, srcq, dstq, out, acc, srcb, dstb, rowb, zbuf, gsem, ssem):
        c = lax.axis_index("c")
        s = lax.axis_index("s")
        _fill_zeros(zbuf, ZR, w2)
        for k in range(TROWS // ZR):
            pltpu.sync_copy(zbuf, acc.at[pl.ds(s * TROWS + k * ZR, ZR)])
        plsc.subcore_barrier()

        vc = vh.at[c]
        base = s * PT_ROWS

        def body(g, carry):
            r0 = base + g * pg
            pltpu.sync_copy(srcq.at[pl.ds(r0, pg)], srcb)
            pltpu.sync_copy(dstq.at[pl.ds(r0, pg)], dstb)
            gcs = [pltpu.async_copy(vc.at[srcb.at[j]], rowb.at[j], gsem)
                   for j in range(pg)]
            for cp in gcs:
                cp.wait()
            scs = [pltpu.async_copy(rowb.at[j], acc.at[dstb.at[j]], ssem,
                                    add=True)
                   for j in range(pg)]
            for cp in scs:
                cp.wait()
            return carry

        lax.fori_loop(0, PT_ROWS // pg, body, 0)
        plsc.subcore_barrier()
        pltpu.sync_copy(acc.at[pl.ds(s * TROWS, TROWS)],
                        out.at[c, pl.ds(s * TROWS, TROWS)])

    return prop


_prop16 = _make_prop(16, 8)
_prop32 = _make_prop(32, 4)


def _rows(i):
    return lax.broadcasted_iota(jnp.int32, (BN, 1), 0) + i * BN


def _t0_body(xp_ref, deg_ref, wf_ref, bf_ref, dinv_ref, v_ref):
    i = pl.program_id(0)
    d = (deg_ref[0] + deg_ref[1] + 1.0)[:, None]
    dv = jnp.where(_rows(i) < N, lax.rsqrt(d), 0.0)
    h = jnp.dot(xp_ref[...], wf_ref[...],
                preferred_element_type=jnp.float32) + bf_ref[...]
    v = dv * h
    dinv_ref[...] = dv
    v_ref[0] = v[:, :16]
    v_ref[1] = v[:, 16:]


def _t1_body(acc_ref, v_ref, dinv_ref, w1_ref, b1_ref, w2_ref, o_ref):
    dv = dinv_ref[...]
    g = jnp.concatenate([acc_ref[0] + v_ref[0], acc_ref[1] + v_ref[1]],
                        axis=1) * dv
    z = jnp.maximum(jnp.dot(g, w1_ref[...],
                            preferred_element_type=jnp.float32) + b1_ref[...],
                    0.0)
    u = jnp.dot(z, w2_ref[...], preferred_element_type=jnp.float32)
    vv = dv * u
    o_ref[0] = vv[:, :32]
    o_ref[1] = vv[:, 32:]


def _t2_body(acc_ref, v_ref, dinv_ref, b2_ref, w3_ref, o_ref):
    dv = dinv_ref[...]
    su = jnp.concatenate([acc_ref[0] + v_ref[0], acc_ref[1] + v_ref[1]],
                         axis=1) * dv
    z = jnp.maximum(su + b2_ref[...], 0.0)
    u = jnp.dot(z, w3_ref[...], preferred_element_type=jnp.float32)
    vv = dv * u
    o_ref[0] = vv[:, :16]
    o_ref[1] = vv[:, 16:]


def _t3_body(acc_ref, v_ref, dinv_ref, b3_ref, o_ref):
    dv = dinv_ref[...]
    s3 = jnp.concatenate([acc_ref[0] + v_ref[0], acc_ref[1] + v_ref[1]],
                         axis=1) * dv
    h = jnp.maximum(s3 + b3_ref[...], 0.0)
    vv = dv * h
    o_ref[0] = vv[:, :16]
    o_ref[1] = vv[:, 16:]


def _tf_body(acc_ref, v_ref, dinv_ref, b3_ref, wo_ref, bo_ref, o_ref):
    dv = dinv_ref[...]
    s3 = jnp.concatenate([acc_ref[0] + v_ref[0], acc_ref[1] + v_ref[1]],
                         axis=1) * dv
    h = jnp.maximum(s3 + b3_ref[...], 0.0)
    o_ref[...] = jnp.dot(h, wo_ref[...],
                         preferred_element_type=jnp.float32) + bo_ref[...]


def _halves(w2):
    return pl.BlockSpec((2, BN, w2), lambda i: (0, i, 0))


def _full(shape):
    nd = len(shape)
    return pl.BlockSpec(shape, lambda i: (0,) * nd)


_DINV = pl.BlockSpec((BN, 1), lambda i: (i, 0))

_t0 = pl.pallas_call(
    _t0_body,
    out_shape=(jax.ShapeDtypeStruct((NPAD, 1), jnp.float32),
               jax.ShapeDtypeStruct((2, NPAD, 16), jnp.float32)),
    grid=(GRID,),
    in_specs=[pl.BlockSpec((BN, 3), lambda i: (i, 0)),
              pl.BlockSpec((2, BN), lambda i: (0, i)),
              _full((3, 32)), _full((1, 32))],
    out_specs=(_DINV, _halves(16)),
)

_t1 = pl.pallas_call(
    _t1_body,
    out_shape=jax.ShapeDtypeStruct((2, NPAD, 32), jnp.float32),
    grid=(GRID,),
    in_specs=[_halves(16), _halves(16), _DINV,
              _full((32, 64)), _full((1, 64)), _full((64, 64))],
    out_specs=_halves(32),
)

_t2 = pl.pallas_call(
    _t2_body,
    out_shape=jax.ShapeDtypeStruct((2, NPAD, 16), jnp.float32),
    grid=(GRID,),
    in_specs=[_halves(32), _halves(32), _DINV,
              _full((1, 64)), _full((64, 32))],
    out_specs=_halves(16),
)

_t3 = pl.pallas_call(
    _t3_body,
    out_shape=jax.ShapeDtypeStruct((2, NPAD, 16), jnp.float32),
    grid=(GRID,),
    in_specs=[_halves(16), _halves(16), _DINV, _full((1, 32))],
    out_specs=_halves(16),
)

_tf = pl.pallas_call(
    _tf_body,
    out_shape=jax.ShapeDtypeStruct((NPAD, 1), jnp.float32),
    grid=(GRID,),
    in_specs=[_halves(16), _halves(16), _DINV,
              _full((1, 32)), _full((32, 1)), _full((1, 1))],
    out_specs=pl.BlockSpec((BN, 1), lambda i: (i, 0)),
)


def kernel(x, edge_index, W_fc1, b_fc1, W_c1, b_c1, W_c2, b_c2, W_c3, b_c3,
           W_fc2, b_fc2):
    src = edge_index[0]
    dst = edge_index[1]
    pad = jnp.full((EPAD - E,), N, jnp.int32)
    srcq = jnp.concatenate([src, pad]).reshape(EROWS, 128)
    dstq = jnp.concatenate([dst, pad]).reshape(EROWS, 128)
    xp = jnp.zeros((NPAD, 3), jnp.float32).at[:N].set(x)

    deg = _deg_pass(dstq)
    dinv, v = _t0(xp, deg, W_fc1, b_fc1.reshape(1, 32))

    b1 = b_c1.reshape(1, 64)
    b2 = b_c2.reshape(1, 64)
    b3 = b_c3.reshape(1, 32)
    for it in range(4):
        a = _prop16(v, srcq, dstq)
        v = _t1(a, v, dinv, W_c1, b1, W_c2)
        a = _prop32(v, srcq, dstq)
        v = _t2(a, v, dinv, b2, W_c3)
        a = _prop16(v, srcq, dstq)
        if it < 3:
            v = _t3(a, v, dinv, b3)
    out = _tf(a, v, dinv, b3, W_fc2, b_fc2.reshape(1, 1))
    return out[:N]


# speedup vs baseline: 27.5804x; 1.3196x over previous
import functools

import jax
import jax.numpy as jnp
from jax import lax
from jax.experimental import pallas as pl
from jax.experimental.pallas import tpu as pltpu
from jax.experimental.pallas import tpu_sc as plsc

N = 50000
NPAD = 51200
TROWS = NPAD // 16
ZR = 200
E = 1600000
EROWS = 12544
EPAD = EROWS * 128
G = 8
PT_ROWS = EROWS // 16
DG_ROWS = EROWS // 32
DG_GROUPS = DG_ROWS // G
BN = 3200
GRID = NPAD // BN

_MESH = plsc.VectorSubcoreMesh(core_axis_name="c", subcore_axis_name="s")
_SC_PARAMS = pltpu.CompilerParams(use_tc_tiling_on_sc=False)


def _fill_zeros(ref, rows, cols):
    z = jnp.zeros((16,), jnp.float32)
    for r in range(rows):
        for k in range(cols // 16):
            ref[r, pl.ds(k * 16, 16)] = z


@functools.partial(
    pl.kernel,
    out_type=jax.ShapeDtypeStruct((2, NPAD), jnp.float32),
    mesh=_MESH,
    scratch_types=[
        pltpu.VMEM_SHARED((NPAD,), jnp.float32),
        pltpu.VMEM((G, 128), jnp.int32),
        pltpu.VMEM((128,), jnp.float32),
        pltpu.VMEM((TROWS,), jnp.float32),
        pltpu.SemaphoreType.DMA,
    ],
    compiler_params=_SC_PARAMS,
)
def _deg_pass(dstq, out, acc, dstb, ones, zbuf, ssem):
    c = lax.axis_index("c")
    s = lax.axis_index("s")
    one = jnp.ones((16,), jnp.float32)
    z = jnp.zeros((16,), jnp.float32)
    for i in range(8):
        ones[pl.ds(i * 16, 16)] = one
    for i in range(TROWS // 16):
        zbuf[pl.ds(i * 16, 16)] = z
    pltpu.sync_copy(zbuf, acc.at[pl.ds(s * TROWS, TROWS)])
    plsc.subcore_barrier()

    base = (s * 2 + c) * DG_ROWS

    def body(g, carry):
        r0 = base + g * G
        pltpu.sync_copy(dstq.at[pl.ds(r0, G)], dstb)
        cps = [pltpu.async_copy(ones, acc.at[dstb.at[j]], ssem, add=True)
               for j in range(G)]
        for cp in cps:
            cp.wait()
        return carry

    lax.fori_loop(0, DG_GROUPS, body, 0)
    plsc.subcore_barrier()
    pltpu.sync_copy(acc.at[pl.ds(s * TROWS, TROWS)],
                    out.at[c, pl.ds(s * TROWS, TROWS)])


W2 = 16
IG = 8
NS = 16
NBLK = PT_ROWS // IG

_ROWB_BYTES = 128 * W2 * 4


@functools.partial(
    pl.kernel,
    out_type=jax.ShapeDtypeStruct((2, NPAD, W2), jnp.float32),
    mesh=_MESH,
    scratch_types=[
        pltpu.VMEM_SHARED((NPAD, W2), jnp.float32),
        pltpu.VMEM((3, IG, 128), jnp.int32),
        pltpu.VMEM((3, IG, 128), jnp.int32),
        pltpu.VMEM((NS, 128, W2), jnp.float32),
        pltpu.VMEM((ZR, W2), jnp.float32),
        pltpu.SemaphoreType.DMA,
        pltpu.SemaphoreType.DMA,
        pltpu.SemaphoreType.DMA((2,)),
    ],
    name="sc_prop16",
    compiler_params=_SC_PARAMS,
)
def _prop16(vh, srcq, dstq, out, acc, srcb, dstb, rowb, zbuf, isem, gsem,
            ssem):
    c = lax.axis_index("c")
    s = lax.axis_index("s")
    vc = vh.at[c]
    base = s * PT_ROWS

    iload0 = [pltpu.async_copy(srcq.at[pl.ds(base, IG)], srcb.at[0], isem),
              pltpu.async_copy(dstq.at[pl.ds(base, IG)], dstb.at[0], isem)]
    _fill_zeros(zbuf, ZR, W2)
    for k in range(TROWS // ZR):
        pltpu.sync_copy(zbuf, acc.at[pl.ds(s * TROWS + k * ZR, ZR)])
    plsc.subcore_barrier()
    for cp in iload0:
        cp.wait()

    def body(m, carry):
        p2 = lax.rem(m, 2)
        p3 = lax.rem(m, 3)
        q3 = lax.rem(m + 1, 3)
        r1 = base + (m + 1) * IG

        for k in range(IG):
            j = p2 * IG + k

            @pl.when(m >= 2)
            def _():
                pltpu.make_async_copy(rowb.at[j], acc.at[dstb.at[p3, k]],
                                      ssem.at[p2]).wait()

            pltpu.async_copy(vc.at[srcb.at[p3, k]], rowb.at[j], gsem)

        @pl.when(m < NBLK - 1)
        def _():
            pltpu.async_copy(srcq.at[pl.ds(r1, IG)], srcb.at[q3], isem)
            pltpu.async_copy(dstq.at[pl.ds(r1, IG)], dstb.at[q3], isem)

        for k in range(IG):
            pltpu.make_async_copy(vc.at[srcb.at[p3, k]],
                                  rowb.at[p2 * IG + k], gsem).wait()
        for k in range(IG):
            pltpu.async_copy(rowb.at[p2 * IG + k], acc.at[dstb.at[p3, k]],
                             ssem.at[p2], add=True)

        @pl.when(m < NBLK - 1)
        def _():
            pltpu.make_async_copy(srcq.at[pl.ds(r1, IG)], srcb.at[q3],
                                  isem).wait()
            pltpu.make_async_copy(dstq.at[pl.ds(r1, IG)], dstb.at[q3],
                                  isem).wait()

        return carry

    lax.fori_loop(0, NBLK, body, 0)
    for k in range(IG):
        pltpu.make_async_copy(rowb.at[k], acc.at[dstb.at[0, 0]],
                              ssem.at[0]).wait()
    for k in range(IG):
        pltpu.make_async_copy(rowb.at[IG + k], acc.at[dstb.at[1, 0]],
                              ssem.at[1]).wait()
    plsc.subcore_barrier()
    pltpu.sync_copy(acc.at[pl.ds(s * TROWS, TROWS)],
                    out.at[c, pl.ds(s * TROWS, TROWS)])


def _rows(i):
    return lax.broadcasted_iota(jnp.int32, (BN, 1), 0) + i * BN


def _t0_body(xp_ref, deg_ref, wf_ref, bf_ref, dinv_ref, v_ref):
    i = pl.program_id(0)
    d = (deg_ref[0] + deg_ref[1] + 1.0)[:, None]
    dv = jnp.where(_rows(i) < N, lax.rsqrt(d), 0.0)
    h = jnp.dot(xp_ref[...], wf_ref[...],
                preferred_element_type=jnp.float32) + bf_ref[...]
    v = dv * h
    dinv_ref[...] = dv
    v_ref[0] = v[:, :16]
    v_ref[1] = v[:, 16:]


def _t1_body(acc_ref, v_ref, dinv_ref, w1_ref, b1_ref, w2_ref, oa_ref,
             ob_ref):
    dv = dinv_ref[...]
    g = jnp.concatenate([acc_ref[0] + v_ref[0], acc_ref[1] + v_ref[1]],
                        axis=1) * dv
    z = jnp.maximum(jnp.dot(g, w1_ref[...],
                            preferred_element_type=jnp.float32) + b1_ref[...],
                    0.0)
    u = jnp.dot(z, w2_ref[...], preferred_element_type=jnp.float32)
    vv = dv * u
    oa_ref[0] = vv[:, :16]
    oa_ref[1] = vv[:, 16:32]
    ob_ref[0] = vv[:, 32:48]
    ob_ref[1] = vv[:, 48:]


def _t2_body(acca_ref, accb_ref, va_ref, vb_ref, dinv_ref, b2_ref, w3_ref,
             o_ref):
    dv = dinv_ref[...]
    su = jnp.concatenate([acca_ref[0] + va_ref[0], acca_ref[1] + va_ref[1],
                          accb_ref[0] + vb_ref[0], accb_ref[1] + vb_ref[1]],
                         axis=1) * dv
    z = jnp.maximum(su + b2_ref[...], 0.0)
    u = jnp.dot(z, w3_ref[...], preferred_element_type=jnp.float32)
    vv = dv * u
    o_ref[0] = vv[:, :16]
    o_ref[1] = vv[:, 16:]


def _t3_body(acc_ref, v_ref, dinv_ref, b3_ref, o_ref):
    dv = dinv_ref[...]
    s3 = jnp.concatenate([acc_ref[0] + v_ref[0], acc_ref[1] + v_ref[1]],
                         axis=1) * dv
    h = jnp.maximum(s3 + b3_ref[...], 0.0)
    vv = dv * h
    o_ref[0] = vv[:, :16]
    o_ref[1] = vv[:, 16:]


def _tf_body(acc_ref, v_ref, dinv_ref, b3_ref, wo_ref, bo_ref, o_ref):
    dv = dinv_ref[...]
    s3 = jnp.concatenate([acc_ref[0] + v_ref[0], acc_ref[1] + v_ref[1]],
                         axis=1) * dv
    h = jnp.maximum(s3 + b3_ref[...], 0.0)
    o_ref[...] = jnp.dot(h, wo_ref[...],
                         preferred_element_type=jnp.float32) + bo_ref[...]


def _halves(w2):
    return pl.BlockSpec((2, BN, w2), lambda i: (0, i, 0))


def _full(shape):
    nd = len(shape)
    return pl.BlockSpec(shape, lambda i: (0,) * nd)


_DINV = pl.BlockSpec((BN, 1), lambda i: (i, 0))

_t0 = pl.pallas_call(
    _t0_body,
    out_shape=(jax.ShapeDtypeStruct((NPAD, 1), jnp.float32),
               jax.ShapeDtypeStruct((2, NPAD, 16), jnp.float32)),
    grid=(GRID,),
    in_specs=[pl.BlockSpec((BN, 3), lambda i: (i, 0)),
              pl.BlockSpec((2, BN), lambda i: (0, i)),
              _full((3, 32)), _full((1, 32))],
    out_specs=(_DINV, _halves(16)),
)

_t1 = pl.pallas_call(
    _t1_body,
    out_shape=(jax.ShapeDtypeStruct((2, NPAD, 16), jnp.float32),
               jax.ShapeDtypeStruct((2, NPAD, 16), jnp.float32)),
    grid=(GRID,),
    in_specs=[_halves(16), _halves(16), _DINV,
              _full((32, 64)), _full((1, 64)), _full((64, 64))],
    out_specs=(_halves(16), _halves(16)),
)

_t2 = pl.pallas_call(
    _t2_body,
    out_shape=jax.ShapeDtypeStruct((2, NPAD, 16), jnp.float32),
    grid=(GRID,),
    in_specs=[_halves(16), _halves(16), _halves(16), _halves(16), _DINV,
              _full((1, 64)), _full((64, 32))],
    out_specs=_halves(16),
)

_t3 = pl.pallas_call(
    _t3_body,
    out_shape=jax.ShapeDtypeStruct((2, NPAD, 16), jnp.float32),
    grid=(GRID,),
    in_specs=[_halves(16), _halves(16), _DINV, _full((1, 32))],
    out_specs=_halves(16),
)

_tf = pl.pallas_call(
    _tf_body,
    out_shape=jax.ShapeDtypeStruct((NPAD, 1), jnp.float32),
    grid=(GRID,),
    in_specs=[_halves(16), _halves(16), _DINV,
              _full((1, 32)), _full((32, 1)), _full((1, 1))],
    out_specs=pl.BlockSpec((BN, 1), lambda i: (i, 0)),
)


def kernel(x, edge_index, W_fc1, b_fc1, W_c1, b_c1, W_c2, b_c2, W_c3, b_c3,
           W_fc2, b_fc2):
    src = edge_index[0]
    dst = edge_index[1]
    pad = jnp.full((EPAD - E,), N, jnp.int32)
    srcq = jnp.concatenate([src, pad]).reshape(EROWS, 128)
    dstq = jnp.concatenate([dst, pad]).reshape(EROWS, 128)
    xp = jnp.zeros((NPAD, 3), jnp.float32).at[:N].set(x)

    deg = _deg_pass(dstq)
    dinv, v = _t0(xp, deg, W_fc1, b_fc1.reshape(1, 32))

    b1 = b_c1.reshape(1, 64)
    b2 = b_c2.reshape(1, 64)
    b3 = b_c3.reshape(1, 32)
    for it in range(4):
        a = _prop16(v, srcq, dstq)
        va, vb = _t1(a, v, dinv, W_c1, b1, W_c2)
        aa = _prop16(va, srcq, dstq)
        ab = _prop16(vb, srcq, dstq)
        v = _t2(aa, ab, va, vb, dinv, b2, W_c3)
        a = _prop16(v, srcq, dstq)
        if it < 3:
            v = _t3(a, v, dinv, b3)
    out = _tf(a, v, dinv, b3, W_fc2, b_fc2.reshape(1, 1))
    return out[:N]


# speedup vs baseline: 38.3913x; 1.3920x over previous
import functools

import jax
import jax.numpy as jnp
from jax import lax
from jax.experimental import pallas as pl
from jax.experimental.pallas import tpu as pltpu
from jax.experimental.pallas import tpu_sc as plsc

N = 50000
NPAD = 51200
TROWS = NPAD // 16
ZR = 200
E = 1600000
EROWS = 12544
EPAD = EROWS * 128
G = 8
PT_ROWS = EROWS // 16
DG_ROWS = EROWS // 32
DG_GROUPS = DG_ROWS // G
BN = 3200
GRID = NPAD // BN

_MESH = plsc.VectorSubcoreMesh(core_axis_name="c", subcore_axis_name="s")
_SC_PARAMS = pltpu.CompilerParams(use_tc_tiling_on_sc=False)


def _fill_zeros(ref, rows, cols):
    z = jnp.zeros((16,), jnp.float32)
    for r in range(rows):
        for k in range(cols // 16):
            ref[r, pl.ds(k * 16, 16)] = z


@functools.partial(
    pl.kernel,
    out_type=jax.ShapeDtypeStruct((2, NPAD), jnp.float32),
    mesh=_MESH,
    scratch_types=[
        pltpu.VMEM_SHARED((NPAD,), jnp.float32),
        pltpu.VMEM((G, 128), jnp.int32),
        pltpu.VMEM((128,), jnp.float32),
        pltpu.VMEM((TROWS,), jnp.float32),
        pltpu.SemaphoreType.DMA,
    ],
    compiler_params=_SC_PARAMS,
)
def _deg_pass(dstq, out, acc, dstb, ones, zbuf, ssem):
    c = lax.axis_index("c")
    s = lax.axis_index("s")
    one = jnp.ones((16,), jnp.float32)
    z = jnp.zeros((16,), jnp.float32)
    for i in range(8):
        ones[pl.ds(i * 16, 16)] = one
    for i in range(TROWS // 16):
        zbuf[pl.ds(i * 16, 16)] = z
    pltpu.sync_copy(zbuf, acc.at[pl.ds(s * TROWS, TROWS)])
    plsc.subcore_barrier()

    base = (s * 2 + c) * DG_ROWS

    def body(g, carry):
        r0 = base + g * G
        pltpu.sync_copy(dstq.at[pl.ds(r0, G)], dstb)
        cps = [pltpu.async_copy(ones, acc.at[dstb.at[j]], ssem, add=True)
               for j in range(G)]
        for cp in cps:
            cp.wait()
        return carry

    lax.fori_loop(0, DG_GROUPS, body, 0)
    plsc.subcore_barrier()
    pltpu.sync_copy(acc.at[pl.ds(s * TROWS, TROWS)],
                    out.at[c, pl.ds(s * TROWS, TROWS)])


W2 = 16
IG = 8
NS = 16
NBLK = PT_ROWS // IG

_ROWB_BYTES = 128 * W2 * 4


@functools.partial(
    pl.kernel,
    out_type=jax.ShapeDtypeStruct((2, NPAD, W2), jnp.float32),
    mesh=_MESH,
    scratch_types=[
        pltpu.VMEM_SHARED((NPAD, W2), jnp.float32),
        pltpu.VMEM((3, IG, 128), jnp.int32),
        pltpu.VMEM((3, IG, 128), jnp.int32),
        pltpu.VMEM((NS, 128, W2), jnp.float32),
        pltpu.VMEM((ZR, W2), jnp.float32),
        pltpu.SemaphoreType.DMA,
        pltpu.SemaphoreType.DMA,
        pltpu.SemaphoreType.DMA((2,)),
    ],
    name="sc_prop16",
    compiler_params=_SC_PARAMS,
)
def _prop16(vh, srcq, dstq, out, acc, srcb, dstb, rowb, zbuf, isem, gsem,
            ssem):
    c = lax.axis_index("c")
    s = lax.axis_index("s")
    vc = vh.at[c]
    base = s * PT_ROWS

    iload0 = [pltpu.async_copy(srcq.at[pl.ds(base, IG)], srcb.at[0], isem),
              pltpu.async_copy(dstq.at[pl.ds(base, IG)], dstb.at[0], isem)]
    _fill_zeros(zbuf, ZR, W2)
    for k in range(TROWS // ZR):
        pltpu.sync_copy(zbuf, acc.at[pl.ds(s * TROWS + k * ZR, ZR)])
    plsc.subcore_barrier()
    for cp in iload0:
        cp.wait()

    def body(m, carry):
        p2 = lax.rem(m, 2)
        p3 = lax.rem(m, 3)
        q3 = lax.rem(m + 1, 3)
        r1 = base + (m + 1) * IG

        for k in range(IG):
            j = p2 * IG + k

            @pl.when(m >= 2)
            def _():
                pltpu.make_async_copy(rowb.at[j], acc.at[dstb.at[p3, k]],
                                      ssem.at[p2]).wait()

            pltpu.async_copy(vc.at[srcb.at[p3, k]], rowb.at[j], gsem)

        @pl.when(m < NBLK - 1)
        def _():
            pltpu.async_copy(srcq.at[pl.ds(r1, IG)], srcb.at[q3], isem)
            pltpu.async_copy(dstq.at[pl.ds(r1, IG)], dstb.at[q3], isem)

        for k in range(IG):
            pltpu.make_async_copy(vc.at[srcb.at[p3, k]],
                                  rowb.at[p2 * IG + k], gsem).wait()
        for k in range(IG):
            pltpu.async_copy(rowb.at[p2 * IG + k], acc.at[dstb.at[p3, k]],
                             ssem.at[p2], add=True)

        @pl.when(m < NBLK - 1)
        def _():
            pltpu.make_async_copy(srcq.at[pl.ds(r1, IG)], srcb.at[q3],
                                  isem).wait()
            pltpu.make_async_copy(dstq.at[pl.ds(r1, IG)], dstb.at[q3],
                                  isem).wait()

        return carry

    lax.fori_loop(0, NBLK, body, 0)
    for k in range(IG):
        pltpu.make_async_copy(rowb.at[k], acc.at[dstb.at[0, 0]],
                              ssem.at[0]).wait()
    for k in range(IG):
        pltpu.make_async_copy(rowb.at[IG + k], acc.at[dstb.at[1, 0]],
                              ssem.at[1]).wait()
    plsc.subcore_barrier()
    pltpu.sync_copy(acc.at[pl.ds(s * TROWS, TROWS)],
                    out.at[c, pl.ds(s * TROWS, TROWS)])


NR = NPAD // 8
BR = NR // GRID
DR = NPAD // 128

import numpy as _np

_SBC = _np.zeros((128, 2048), _np.float32)
for _j in range(16):
    for _i in range(8):
        _SBC[8 * _j + _i, 128 * _j + 16 * _i:128 * _j + 16 * _i + 16] = 1.0


def _t0_body(xs_ref, deg_ref, sbc_ref, k0_ref, b0_ref, dscr_ref, v_ref):
    d = deg_ref[0] + deg_ref[1] + 1.0
    node = (lax.broadcasted_iota(jnp.int32, (DR, 128), 0) * 128
            + lax.broadcasted_iota(jnp.int32, (DR, 128), 1))
    dv = jnp.where(node < N, lax.rsqrt(d), 0.0)
    dscr_ref[...] = jnp.dot(dv, sbc_ref[...],
                            precision=lax.Precision.HIGHEST,
                            preferred_element_type=jnp.float32)
    ds = dscr_ref[...].reshape(NR, 128)
    xs = xs_ref[...]
    for q in range(2):
        h = jnp.dot(xs, k0_ref[q], preferred_element_type=jnp.float32)
        v_ref[q] = ds * (h + b0_ref[q])


def _t1_body(acc_ref, v_ref, ds_ref, k1_ref, b1_ref, k2_ref, oa_ref, ob_ref):
    ds = ds_ref[...]
    g = [ds * (acc_ref[q] + v_ref[q]) for q in range(2)]
    z = [jnp.maximum(
            jnp.dot(g[0], k1_ref[0, q], preferred_element_type=jnp.float32)
            + jnp.dot(g[1], k1_ref[1, q], preferred_element_type=jnp.float32)
            + b1_ref[q], 0.0)
         for q in range(4)]
    for qq in range(4):
        u = jnp.dot(z[0], k2_ref[0, qq], preferred_element_type=jnp.float32)
        for q in range(1, 4):
            u = u + jnp.dot(z[q], k2_ref[q, qq],
                            preferred_element_type=jnp.float32)
        o = ds * u
        if qq < 2:
            oa_ref[qq] = o
        else:
            ob_ref[qq - 2] = o


def _t2_body(acca_ref, accb_ref, va_ref, vb_ref, ds_ref, b2_ref, k3_ref,
             o_ref):
    ds = ds_ref[...]
    z = [jnp.maximum(ds * (acca_ref[q] + va_ref[q]) + b2_ref[q], 0.0)
         for q in range(2)]
    z += [jnp.maximum(ds * (accb_ref[q] + vb_ref[q]) + b2_ref[q + 2], 0.0)
          for q in range(2)]
    for qq in range(2):
        u = jnp.dot(z[0], k3_ref[0, qq], preferred_element_type=jnp.float32)
        for q in range(1, 4):
            u = u + jnp.dot(z[q], k3_ref[q, qq],
                            preferred_element_type=jnp.float32)
        o_ref[qq] = ds * u


def _t3_body(acc_ref, v_ref, ds_ref, b3_ref, o_ref):
    ds = ds_ref[...]
    for q in range(2):
        h = jnp.maximum(ds * (acc_ref[q] + v_ref[q]) + b3_ref[q], 0.0)
        o_ref[q] = ds * h


def _tf_body(acc_ref, v_ref, ds_ref, b3_ref, kf_ref, bo_ref, o_ref):
    ds = ds_ref[...]
    h = [jnp.maximum(ds * (acc_ref[q] + v_ref[q]) + b3_ref[q], 0.0)
         for q in range(2)]
    o_ref[...] = (jnp.dot(h[0], kf_ref[0], preferred_element_type=jnp.float32)
                  + jnp.dot(h[1], kf_ref[1],
                            preferred_element_type=jnp.float32)
                  + bo_ref[0, 0])


def _scr(n):
    return pl.BlockSpec((n, BR, 128), lambda i: (0, i, 0))


def _full(shape):
    nd = len(shape)
    return pl.BlockSpec(shape, lambda i: (0,) * nd)


_DS = pl.BlockSpec((BR, 128), lambda i: (i, 0))

_t0 = pl.pallas_call(
    _t0_body,
    out_shape=(jax.ShapeDtypeStruct((DR, 2048), jnp.float32),
               jax.ShapeDtypeStruct((2, NR, 128), jnp.float32)),
    grid=(1,),
    in_specs=[_full((NR, 128)), _full((2, DR, 128)), _full((128, 2048)),
              _full((2, 128, 128)), _full((2, 1, 128))],
    out_specs=(_full((DR, 2048)), _full((2, NR, 128))),
)

_t1 = pl.pallas_call(
    _t1_body,
    out_shape=(jax.ShapeDtypeStruct((2, NR, 128), jnp.float32),
               jax.ShapeDtypeStruct((2, NR, 128), jnp.float32)),
    grid=(GRID,),
    in_specs=[_scr(2), _scr(2), _DS,
              _full((2, 4, 128, 128)), _full((4, 1, 128)),
              _full((4, 4, 128, 128))],
    out_specs=(_scr(2), _scr(2)),
)

_t2 = pl.pallas_call(
    _t2_body,
    out_shape=jax.ShapeDtypeStruct((2, NR, 128), jnp.float32),
    grid=(GRID,),
    in_specs=[_scr(2), _scr(2), _scr(2), _scr(2), _DS,
              _full((4, 1, 128)), _full((4, 2, 128, 128))],
    out_specs=_scr(2),
)

_t3 = pl.pallas_call(
    _t3_body,
    out_shape=jax.ShapeDtypeStruct((2, NR, 128), jnp.float32),
    grid=(GRID,),
    in_specs=[_scr(2), _scr(2), _DS, _full((2, 1, 128))],
    out_specs=_scr(2),
)

_tf = pl.pallas_call(
    _tf_body,
    out_shape=jax.ShapeDtypeStruct((NR, 8), jnp.float32),
    grid=(GRID,),
    in_specs=[_scr(2), _scr(2), _DS, _full((2, 1, 128)),
              _full((2, 128, 8)), _full((1, 1))],
    out_specs=pl.BlockSpec((BR, 8), lambda i: (i, 0)),
)


_I8 = _np.eye(8, dtype=_np.float32)


def _kron_blocks(w, nq_in, nq_out):
    blk = w.reshape(nq_in, 16, nq_out, 16).transpose(0, 2, 1, 3)
    return jnp.einsum('ij,qpab->qpiajb', _I8, blk).reshape(
        nq_in, nq_out, 128, 128)


def _bias_scr(b, nq):
    return jnp.tile(b.reshape(nq, 1, 16), (1, 1, 8)).reshape(nq, 1, 128)


def _as_sc(v):
    return v.reshape(2, NPAD, W2)


def _as_tc(a):
    return a.reshape(2, NR, 128)


def kernel(x, edge_index, W_fc1, b_fc1, W_c1, b_c1, W_c2, b_c2, W_c3, b_c3,
           W_fc2, b_fc2):
    src = edge_index[0]
    dst = edge_index[1]
    pad = jnp.full((EPAD - E,), N, jnp.int32)
    srcq = jnp.concatenate([src, pad]).reshape(EROWS, 128)
    dstq = jnp.concatenate([dst, pad]).reshape(EROWS, 128)
    xs = jnp.zeros((NPAD, 16), jnp.float32).at[:N, :3].set(x).reshape(NR, 128)

    wf16 = jnp.zeros((16, 32), jnp.float32).at[:3].set(W_fc1)
    k0 = _kron_blocks(wf16, 1, 2)[0]
    k1 = _kron_blocks(W_c1, 2, 4)
    k2 = _kron_blocks(W_c2, 4, 4)
    k3 = _kron_blocks(W_c3, 4, 2)
    kfq = W_fc2.reshape(2, 16, 1)
    kf = jnp.einsum('ij,qab->qiajb', _I8, kfq).reshape(2, 128, 8)
    b0 = _bias_scr(b_fc1, 2)
    b1 = _bias_scr(b_c1, 4)
    b2 = _bias_scr(b_c2, 4)
    b3 = _bias_scr(b_c3, 2)

    deg = _deg_pass(dstq)
    dscr, v = _t0(xs, deg.reshape(2, DR, 128), _SBC, k0, b0)
    ds = dscr.reshape(NR, 128)

    for it in range(4):
        a = _prop16(_as_sc(v), srcq, dstq)
        va, vb = _t1(_as_tc(a), v, ds, k1, b1, k2)
        aa = _prop16(_as_sc(va), srcq, dstq)
        ab = _prop16(_as_sc(vb), srcq, dstq)
        v = _t2(_as_tc(aa), _as_tc(ab), va, vb, ds, b2, k3)
        a = _prop16(_as_sc(v), srcq, dstq)
        if it < 3:
            v = _t3(_as_tc(a), v, ds, b3)
    out = _tf(_as_tc(a), v, ds, b3, kf, b_fc2.reshape(1, 1))
    return out.reshape(NPAD, 1)[:N]


# speedup vs baseline: 42.9414x; 1.1185x over previous
import functools

import jax
import jax.numpy as jnp
from jax import lax
from jax.experimental import pallas as pl
from jax.experimental.pallas import tpu as pltpu
from jax.experimental.pallas import tpu_sc as plsc

N = 50000
NPAD = 51200
TROWS = NPAD // 16
ZR = 200
E = 1600000
EROWS = 12544
EPAD = EROWS * 128
G = 8
PT_ROWS = EROWS // 16
DG_ROWS = EROWS // 32
DG_GROUPS = DG_ROWS // G
BN = 3200
GRID = NPAD // BN

_MESH = plsc.VectorSubcoreMesh(core_axis_name="c", subcore_axis_name="s")
_SC_PARAMS = pltpu.CompilerParams(use_tc_tiling_on_sc=False)


def _fill_zeros(ref, rows, cols):
    z = jnp.zeros((16,), jnp.float32)
    for r in range(rows):
        for k in range(cols // 16):
            ref[r, pl.ds(k * 16, 16)] = z


@functools.partial(
    pl.kernel,
    out_type=jax.ShapeDtypeStruct((2, NPAD), jnp.float32),
    mesh=_MESH,
    scratch_types=[
        pltpu.VMEM_SHARED((NPAD,), jnp.float32),
        pltpu.VMEM((G, 128), jnp.int32),
        pltpu.VMEM((128,), jnp.float32),
        pltpu.VMEM((TROWS,), jnp.float32),
        pltpu.SemaphoreType.DMA,
    ],
    compiler_params=_SC_PARAMS,
)
def _deg_pass(dstq, out, acc, dstb, ones, zbuf, ssem):
    c = lax.axis_index("c")
    s = lax.axis_index("s")
    one = jnp.ones((16,), jnp.float32)
    z = jnp.zeros((16,), jnp.float32)
    for i in range(8):
        ones[pl.ds(i * 16, 16)] = one
    for i in range(TROWS // 16):
        zbuf[pl.ds(i * 16, 16)] = z
    pltpu.sync_copy(zbuf, acc.at[pl.ds(s * TROWS, TROWS)])
    plsc.subcore_barrier()

    base = (s * 2 + c) * DG_ROWS

    def body(g, carry):
        r0 = base + g * G
        pltpu.sync_copy(dstq.at[pl.ds(r0, G)], dstb)
        cps = [pltpu.async_copy(ones, acc.at[dstb.at[j]], ssem, add=True)
               for j in range(G)]
        for cp in cps:
            cp.wait()
        return carry

    lax.fori_loop(0, DG_GROUPS, body, 0)
    plsc.subcore_barrier()
    pltpu.sync_copy(acc.at[pl.ds(s * TROWS, TROWS)],
                    out.at[c, pl.ds(s * TROWS, TROWS)])


W2 = 16
IG = 14
NS = 28
NBLK = PT_ROWS // IG

_ROWB_BYTES = 128 * W2 * 4


@functools.partial(
    pl.kernel,
    out_type=jax.ShapeDtypeStruct((2, NPAD, W2), jnp.float32),
    mesh=_MESH,
    scratch_types=[
        pltpu.VMEM_SHARED((NPAD, W2), jnp.float32),
        pltpu.VMEM((3, IG, 128), jnp.int32),
        pltpu.VMEM((3, IG, 128), jnp.int32),
        pltpu.VMEM((NS, 128, W2), jnp.float32),
        pltpu.VMEM((ZR, W2), jnp.float32),
        pltpu.SemaphoreType.DMA,
        pltpu.SemaphoreType.DMA,
        pltpu.SemaphoreType.DMA((2,)),
    ],
    name="sc_prop16",
    compiler_params=_SC_PARAMS,
)
def _prop16(vh, srcq, dstq, out, acc, srcb, dstb, rowb, zbuf, isem, gsem,
            ssem):
    c = lax.axis_index("c")
    s = lax.axis_index("s")
    vc = vh.at[c]
    base = s * PT_ROWS

    iload0 = [pltpu.async_copy(srcq.at[pl.ds(base, IG)], srcb.at[0], isem),
              pltpu.async_copy(dstq.at[pl.ds(base, IG)], dstb.at[0], isem)]
    _fill_zeros(zbuf, ZR, W2)
    for k in range(TROWS // ZR):
        pltpu.sync_copy(zbuf, acc.at[pl.ds(s * TROWS + k * ZR, ZR)])
    plsc.subcore_barrier()
    for cp in iload0:
        cp.wait()

    def body(m, carry):
        p2 = lax.rem(m, 2)
        p3 = lax.rem(m, 3)
        q3 = lax.rem(m + 1, 3)
        r1 = base + (m + 1) * IG

        for k in range(IG):
            j = p2 * IG + k

            @pl.when(m >= 2)
            def _():
                pltpu.make_async_copy(rowb.at[j], acc.at[dstb.at[p3, k]],
                                      ssem.at[p2]).wait()

            pltpu.async_copy(vc.at[srcb.at[p3, k]], rowb.at[j], gsem)

        @pl.when(m < NBLK - 1)
        def _():
            pltpu.async_copy(srcq.at[pl.ds(r1, IG)], srcb.at[q3], isem)
            pltpu.async_copy(dstq.at[pl.ds(r1, IG)], dstb.at[q3], isem)

        for k in range(IG):
            pltpu.make_async_copy(vc.at[srcb.at[p3, k]],
                                  rowb.at[p2 * IG + k], gsem).wait()
        for k in range(IG):
            pltpu.async_copy(rowb.at[p2 * IG + k], acc.at[dstb.at[p3, k]],
                             ssem.at[p2], add=True)

        @pl.when(m < NBLK - 1)
        def _():
            pltpu.make_async_copy(srcq.at[pl.ds(r1, IG)], srcb.at[q3],
                                  isem).wait()
            pltpu.make_async_copy(dstq.at[pl.ds(r1, IG)], dstb.at[q3],
                                  isem).wait()

        return carry

    lax.fori_loop(0, NBLK, body, 0)
    for k in range(IG):
        pltpu.make_async_copy(rowb.at[k], acc.at[dstb.at[0, 0]],
                              ssem.at[0]).wait()
    for k in range(IG):
        pltpu.make_async_copy(rowb.at[IG + k], acc.at[dstb.at[1, 0]],
                              ssem.at[1]).wait()
    plsc.subcore_barrier()
    pltpu.sync_copy(acc.at[pl.ds(s * TROWS, TROWS)],
                    out.at[c, pl.ds(s * TROWS, TROWS)])


NR = NPAD // 8
BR = NR // GRID
DR = NPAD // 128

import numpy as _np

_SBC = _np.zeros((128, 2048), _np.float32)
for _j in range(16):
    for _i in range(8):
        _SBC[8 * _j + _i, 128 * _j + 16 * _i:128 * _j + 16 * _i + 16] = 1.0


def _t0_body(xs_ref, deg_ref, sbc_ref, k0_ref, b0_ref, dscr_ref, v_ref):
    d = deg_ref[0] + deg_ref[1] + 1.0
    node = (lax.broadcasted_iota(jnp.int32, (DR, 128), 0) * 128
            + lax.broadcasted_iota(jnp.int32, (DR, 128), 1))
    dv = jnp.where(node < N, lax.rsqrt(d), 0.0)
    dscr_ref[...] = jnp.dot(dv, sbc_ref[...],
                            precision=lax.Precision.HIGHEST,
                            preferred_element_type=jnp.float32)
    ds = dscr_ref[...].reshape(NR, 128)
    xs = xs_ref[...]
    for q in range(2):
        h = jnp.dot(xs, k0_ref[q], preferred_element_type=jnp.float32)
        v_ref[q] = ds * (h + b0_ref[q])


def _t1_body(acc_ref, v_ref, ds_ref, k1_ref, b1_ref, k2_ref, oa_ref, ob_ref):
    ds = ds_ref[...]
    g = [ds * (acc_ref[q] + v_ref[q]) for q in range(2)]
    z = [jnp.maximum(
            jnp.dot(g[0], k1_ref[0, q], preferred_element_type=jnp.float32)
            + jnp.dot(g[1], k1_ref[1, q], preferred_element_type=jnp.float32)
            + b1_ref[q], 0.0)
         for q in range(4)]
    for qq in range(4):
        u = jnp.dot(z[0], k2_ref[0, qq], preferred_element_type=jnp.float32)
        for q in range(1, 4):
            u = u + jnp.dot(z[q], k2_ref[q, qq],
                            preferred_element_type=jnp.float32)
        o = ds * u
        if qq < 2:
            oa_ref[qq] = o
        else:
            ob_ref[qq - 2] = o


def _t2_body(acca_ref, accb_ref, va_ref, vb_ref, ds_ref, b2_ref, k3_ref,
             o_ref):
    ds = ds_ref[...]
    z = [jnp.maximum(ds * (acca_ref[q] + va_ref[q]) + b2_ref[q], 0.0)
         for q in range(2)]
    z += [jnp.maximum(ds * (accb_ref[q] + vb_ref[q]) + b2_ref[q + 2], 0.0)
          for q in range(2)]
    for qq in range(2):
        u = jnp.dot(z[0], k3_ref[0, qq], preferred_element_type=jnp.float32)
        for q in range(1, 4):
            u = u + jnp.dot(z[q], k3_ref[q, qq],
                            preferred_element_type=jnp.float32)
        o_ref[qq] = ds * u


def _t3_body(acc_ref, v_ref, ds_ref, b3_ref, o_ref):
    ds = ds_ref[...]
    for q in range(2):
        h = jnp.maximum(ds * (acc_ref[q] + v_ref[q]) + b3_ref[q], 0.0)
        o_ref[q] = ds * h


def _tf_body(acc_ref, v_ref, ds_ref, b3_ref, kf_ref, bo_ref, o_ref):
    ds = ds_ref[...]
    h = [jnp.maximum(ds * (acc_ref[q] + v_ref[q]) + b3_ref[q], 0.0)
         for q in range(2)]
    o_ref[...] = (jnp.dot(h[0], kf_ref[0], preferred_element_type=jnp.float32)
                  + jnp.dot(h[1], kf_ref[1],
                            preferred_element_type=jnp.float32)
                  + bo_ref[0, 0])


def _scr(n):
    return pl.BlockSpec((n, BR, 128), lambda i: (0, i, 0))


def _full(shape):
    nd = len(shape)
    return pl.BlockSpec(shape, lambda i: (0,) * nd)


_DS = pl.BlockSpec((BR, 128), lambda i: (i, 0))

_t0 = pl.pallas_call(
    _t0_body,
    out_shape=(jax.ShapeDtypeStruct((DR, 2048), jnp.float32),
               jax.ShapeDtypeStruct((2, NR, 128), jnp.float32)),
    grid=(1,),
    in_specs=[_full((NR, 128)), _full((2, DR, 128)), _full((128, 2048)),
              _full((2, 128, 128)), _full((2, 1, 128))],
    out_specs=(_full((DR, 2048)), _full((2, NR, 128))),
)

_t1 = pl.pallas_call(
    _t1_body,
    out_shape=(jax.ShapeDtypeStruct((2, NR, 128), jnp.float32),
               jax.ShapeDtypeStruct((2, NR, 128), jnp.float32)),
    grid=(GRID,),
    in_specs=[_scr(2), _scr(2), _DS,
              _full((2, 4, 128, 128)), _full((4, 1, 128)),
              _full((4, 4, 128, 128))],
    out_specs=(_scr(2), _scr(2)),
)

_t2 = pl.pallas_call(
    _t2_body,
    out_shape=jax.ShapeDtypeStruct((2, NR, 128), jnp.float32),
    grid=(GRID,),
    in_specs=[_scr(2), _scr(2), _scr(2), _scr(2), _DS,
              _full((4, 1, 128)), _full((4, 2, 128, 128))],
    out_specs=_scr(2),
)

_t3 = pl.pallas_call(
    _t3_body,
    out_shape=jax.ShapeDtypeStruct((2, NR, 128), jnp.float32),
    grid=(GRID,),
    in_specs=[_scr(2), _scr(2), _DS, _full((2, 1, 128))],
    out_specs=_scr(2),
)

_tf = pl.pallas_call(
    _tf_body,
    out_shape=jax.ShapeDtypeStruct((NR, 8), jnp.float32),
    grid=(GRID,),
    in_specs=[_scr(2), _scr(2), _DS, _full((2, 1, 128)),
              _full((2, 128, 8)), _full((1, 1))],
    out_specs=pl.BlockSpec((BR, 8), lambda i: (i, 0)),
)


_I8 = _np.eye(8, dtype=_np.float32)


def _kron_blocks(w, nq_in, nq_out):
    blk = w.reshape(nq_in, 16, nq_out, 16).transpose(0, 2, 1, 3)
    return jnp.einsum('ij,qpab->qpiajb', _I8, blk).reshape(
        nq_in, nq_out, 128, 128)


def _bias_scr(b, nq):
    return jnp.tile(b.reshape(nq, 1, 16), (1, 1, 8)).reshape(nq, 1, 128)


def _as_sc(v):
    return v.reshape(2, NPAD, W2)


def _as_tc(a):
    return a.reshape(2, NR, 128)


def kernel(x, edge_index, W_fc1, b_fc1, W_c1, b_c1, W_c2, b_c2, W_c3, b_c3,
           W_fc2, b_fc2):
    src = edge_index[0]
    dst = edge_index[1]
    pad = jnp.full((EPAD - E,), N, jnp.int32)
    srcq = jnp.concatenate([src, pad]).reshape(EROWS, 128)
    dstq = jnp.concatenate([dst, pad]).reshape(EROWS, 128)
    xs = jnp.zeros((NPAD, 16), jnp.float32).at[:N, :3].set(x).reshape(NR, 128)

    wf16 = jnp.zeros((16, 32), jnp.float32).at[:3].set(W_fc1)
    k0 = _kron_blocks(wf16, 1, 2)[0]
    k1 = _kron_blocks(W_c1, 2, 4)
    k2 = _kron_blocks(W_c2, 4, 4)
    k3 = _kron_blocks(W_c3, 4, 2)
    kfq = W_fc2.reshape(2, 16, 1)
    kf = jnp.einsum('ij,qab->qiajb', _I8, kfq).reshape(2, 128, 8)
    b0 = _bias_scr(b_fc1, 2)
    b1 = _bias_scr(b_c1, 4)
    b2 = _bias_scr(b_c2, 4)
    b3 = _bias_scr(b_c3, 2)

    deg = _deg_pass(dstq)
    dscr, v = _t0(xs, deg.reshape(2, DR, 128), _SBC, k0, b0)
    ds = dscr.reshape(NR, 128)

    for it in range(4):
        a = _prop16(_as_sc(v), srcq, dstq)
        va, vb = _t1(_as_tc(a), v, ds, k1, b1, k2)
        aa = _prop16(_as_sc(va), srcq, dstq)
        ab = _prop16(_as_sc(vb), srcq, dstq)
        v = _t2(_as_tc(aa), _as_tc(ab), va, vb, ds, b2, k3)
        a = _prop16(_as_sc(v), srcq, dstq)
        if it < 3:
            v = _t3(_as_tc(a), v, ds, b3)
    out = _tf(_as_tc(a), v, ds, b3, kf, b_fc2.reshape(1, 1))
    return out.reshape(NPAD, 1)[:N]


# speedup vs baseline: 49.9427x; 1.1630x over previous
import functools

import jax
import jax.numpy as jnp
from jax import lax
from jax.experimental import pallas as pl
from jax.experimental.pallas import tpu as pltpu
from jax.experimental.pallas import tpu_sc as plsc

N = 50000
NPAD = 51200
TROWS = NPAD // 16
ZR = 200
E = 1600000
EROWS = 12544
EPAD = EROWS * 128
G = 8
PT_ROWS = EROWS // 16
DG_ROWS = EROWS // 32
DG_GROUPS = DG_ROWS // G
BN = 3200
GRID = NPAD // BN

_MESH = plsc.VectorSubcoreMesh(core_axis_name="c", subcore_axis_name="s")
_SC_PARAMS = pltpu.CompilerParams(use_tc_tiling_on_sc=False)


def _fill_zeros(ref, rows, cols):
    z = jnp.zeros((16,), jnp.float32)
    for r in range(rows):
        for k in range(cols // 16):
            ref[r, pl.ds(k * 16, 16)] = z


@functools.partial(
    pl.kernel,
    out_type=jax.ShapeDtypeStruct((2, NPAD), jnp.float32),
    mesh=_MESH,
    scratch_types=[
        pltpu.VMEM_SHARED((NPAD,), jnp.float32),
        pltpu.VMEM((G, 128), jnp.int32),
        pltpu.VMEM((128,), jnp.float32),
        pltpu.VMEM((TROWS,), jnp.float32),
        pltpu.SemaphoreType.DMA,
    ],
    compiler_params=_SC_PARAMS,
)
def _deg_pass(dstq, out, acc, dstb, ones, zbuf, ssem):
    c = lax.axis_index("c")
    s = lax.axis_index("s")
    one = jnp.ones((16,), jnp.float32)
    z = jnp.zeros((16,), jnp.float32)
    for i in range(8):
        ones[pl.ds(i * 16, 16)] = one
    for i in range(TROWS // 16):
        zbuf[pl.ds(i * 16, 16)] = z
    pltpu.sync_copy(zbuf, acc.at[pl.ds(s * TROWS, TROWS)])
    plsc.subcore_barrier()

    base = (s * 2 + c) * DG_ROWS

    def body(g, carry):
        r0 = base + g * G
        pltpu.sync_copy(dstq.at[pl.ds(r0, G)], dstb)
        cps = [pltpu.async_copy(ones, acc.at[dstb.at[j]], ssem, add=True)
               for j in range(G)]
        for cp in cps:
            cp.wait()
        return carry

    lax.fori_loop(0, DG_GROUPS, body, 0)
    plsc.subcore_barrier()
    pltpu.sync_copy(acc.at[pl.ds(s * TROWS, TROWS)],
                    out.at[c, pl.ds(s * TROWS, TROWS)])


W2 = 16
IG = 8
NS = 3 * IG
NBLK = PT_ROWS // IG


@functools.partial(
    pl.kernel,
    out_type=jax.ShapeDtypeStruct((2, NPAD, W2), jnp.float32),
    mesh=_MESH,
    scratch_types=[
        pltpu.VMEM_SHARED((NPAD, W2), jnp.float32),
        pltpu.VMEM((4, IG, 128), jnp.int32),
        pltpu.VMEM((4, IG, 128), jnp.int32),
        pltpu.VMEM((NS, 128, W2), jnp.float32),
        pltpu.VMEM((ZR, W2), jnp.float32),
        pltpu.SemaphoreType.DMA,
        pltpu.SemaphoreType.DMA((2,)),
        pltpu.SemaphoreType.DMA((2,)),
    ],
    name="sc_prop16",
    compiler_params=_SC_PARAMS,
)
def _prop16(vh, srcq, dstq, out, acc, srcb, dstb, rowb, zbuf, isem, gsem,
            ssem):
    c = lax.axis_index("c")
    s = lax.axis_index("s")
    vc = vh.at[c]
    base = s * PT_ROWS

    iload0 = [pltpu.async_copy(srcq.at[pl.ds(base, IG)], srcb.at[0], isem),
              pltpu.async_copy(dstq.at[pl.ds(base, IG)], dstb.at[0], isem)]
    _fill_zeros(zbuf, ZR, W2)
    for k in range(TROWS // ZR):
        pltpu.sync_copy(zbuf, acc.at[pl.ds(s * TROWS + k * ZR, ZR)])
    plsc.subcore_barrier()
    for cp in iload0:
        cp.wait()

    def body(m, carry):
        g2 = lax.rem(m, 2)
        g3 = lax.rem(m, 3)
        g4 = lax.rem(m, 4)
        h2 = lax.rem(m + 1, 2)
        h3 = lax.rem(m + 2, 3)
        h4 = lax.rem(m + 3, 4)
        f4 = lax.rem(m + 1, 4)
        r1 = base + (m + 1) * IG

        @pl.when(m >= 3)
        def _():
            for k in range(IG):
                pltpu.make_async_copy(rowb.at[k], acc.at[dstb.at[f4, 0]],
                                      ssem.at[h2]).wait()

        @pl.when(m < NBLK)
        def _():
            for k in range(IG):
                pltpu.async_copy(vc.at[srcb.at[g4, k]],
                                 rowb.at[g3 * IG + k], gsem.at[g2])

        @pl.when(m < NBLK - 1)
        def _():
            pltpu.async_copy(srcq.at[pl.ds(r1, IG)], srcb.at[f4], isem)
            pltpu.async_copy(dstq.at[pl.ds(r1, IG)], dstb.at[f4], isem)

        @pl.when(jnp.logical_and(m >= 1, m <= NBLK))
        def _():
            for k in range(IG):
                pltpu.make_async_copy(vc.at[srcb.at[h4, k]],
                                      rowb.at[h3 * IG + k],
                                      gsem.at[h2]).wait()
            for k in range(IG):
                pltpu.async_copy(rowb.at[h3 * IG + k],
                                 acc.at[dstb.at[h4, k]],
                                 ssem.at[h2], add=True)

        @pl.when(m < NBLK - 1)
        def _():
            pltpu.make_async_copy(srcq.at[pl.ds(r1, IG)], srcb.at[f4],
                                  isem).wait()
            pltpu.make_async_copy(dstq.at[pl.ds(r1, IG)], dstb.at[f4],
                                  isem).wait()

        return carry

    lax.fori_loop(0, NBLK + 3, body, 0)
    plsc.subcore_barrier()
    pltpu.sync_copy(acc.at[pl.ds(s * TROWS, TROWS)],
                    out.at[c, pl.ds(s * TROWS, TROWS)])


NR = NPAD // 8
BR = NR // GRID
DR = NPAD // 128

import numpy as _np

_SBC = _np.zeros((128, 2048), _np.float32)
for _j in range(16):
    for _i in range(8):
        _SBC[8 * _j + _i, 128 * _j + 16 * _i:128 * _j + 16 * _i + 16] = 1.0


def _t0_body(xs_ref, deg_ref, sbc_ref, k0_ref, b0_ref, dscr_ref, v_ref):
    d = deg_ref[0] + deg_ref[1] + 1.0
    node = (lax.broadcasted_iota(jnp.int32, (DR, 128), 0) * 128
            + lax.broadcasted_iota(jnp.int32, (DR, 128), 1))
    dv = jnp.where(node < N, lax.rsqrt(d), 0.0)
    dscr_ref[...] = jnp.dot(dv, sbc_ref[...],
                            precision=lax.Precision.HIGHEST,
                            preferred_element_type=jnp.float32)
    ds = dscr_ref[...].reshape(NR, 128)
    xs = xs_ref[...]
    for q in range(2):
        h = jnp.dot(xs, k0_ref[q], preferred_element_type=jnp.float32)
        v_ref[q] = ds * (h + b0_ref[q])


def _t1_body(acc_ref, v_ref, ds_ref, k1_ref, b1_ref, k2_ref, oa_ref, ob_ref):
    ds = ds_ref[...]
    g = [ds * (acc_ref[q] + v_ref[q]) for q in range(2)]
    z = [jnp.maximum(
            jnp.dot(g[0], k1_ref[0, q], preferred_element_type=jnp.float32)
            + jnp.dot(g[1], k1_ref[1, q], preferred_element_type=jnp.float32)
            + b1_ref[q], 0.0)
         for q in range(4)]
    for qq in range(4):
        u = jnp.dot(z[0], k2_ref[0, qq], preferred_element_type=jnp.float32)
        for q in range(1, 4):
            u = u + jnp.dot(z[q], k2_ref[q, qq],
                            preferred_element_type=jnp.float32)
        o = ds * u
        if qq < 2:
            oa_ref[qq] = o
        else:
            ob_ref[qq - 2] = o


def _t2_body(acca_ref, accb_ref, va_ref, vb_ref, ds_ref, b2_ref, k3_ref,
             o_ref):
    ds = ds_ref[...]
    z = [jnp.maximum(ds * (acca_ref[q] + va_ref[q]) + b2_ref[q], 0.0)
         for q in range(2)]
    z += [jnp.maximum(ds * (accb_ref[q] + vb_ref[q]) + b2_ref[q + 2], 0.0)
          for q in range(2)]
    for qq in range(2):
        u = jnp.dot(z[0], k3_ref[0, qq], preferred_element_type=jnp.float32)
        for q in range(1, 4):
            u = u + jnp.dot(z[q], k3_ref[q, qq],
                            preferred_element_type=jnp.float32)
        o_ref[qq] = ds * u


def _t3_body(acc_ref, v_ref, ds_ref, b3_ref, o_ref):
    ds = ds_ref[...]
    for q in range(2):
        h = jnp.maximum(ds * (acc_ref[q] + v_ref[q]) + b3_ref[q], 0.0)
        o_ref[q] = ds * h


def _tf_body(acc_ref, v_ref, ds_ref, b3_ref, kf_ref, bo_ref, o_ref):
    ds = ds_ref[...]
    h = [jnp.maximum(ds * (acc_ref[q] + v_ref[q]) + b3_ref[q], 0.0)
         for q in range(2)]
    o_ref[...] = (jnp.dot(h[0], kf_ref[0], preferred_element_type=jnp.float32)
                  + jnp.dot(h[1], kf_ref[1],
                            preferred_element_type=jnp.float32)
                  + bo_ref[0, 0])


def _scr(n):
    return pl.BlockSpec((n, BR, 128), lambda i: (0, i, 0))


def _full(shape):
    nd = len(shape)
    return pl.BlockSpec(shape, lambda i: (0,) * nd)


_DS = pl.BlockSpec((BR, 128), lambda i: (i, 0))

_t0 = pl.pallas_call(
    _t0_body,
    out_shape=(jax.ShapeDtypeStruct((DR, 2048), jnp.float32),
               jax.ShapeDtypeStruct((2, NR, 128), jnp.float32)),
    grid=(1,),
    in_specs=[_full((NR, 128)), _full((2, DR, 128)), _full((128, 2048)),
              _full((2, 128, 128)), _full((2, 1, 128))],
    out_specs=(_full((DR, 2048)), _full((2, NR, 128))),
)

_t1 = pl.pallas_call(
    _t1_body,
    out_shape=(jax.ShapeDtypeStruct((2, NR, 128), jnp.float32),
               jax.ShapeDtypeStruct((2, NR, 128), jnp.float32)),
    grid=(GRID,),
    in_specs=[_scr(2), _scr(2), _DS,
              _full((2, 4, 128, 128)), _full((4, 1, 128)),
              _full((4, 4, 128, 128))],
    out_specs=(_scr(2), _scr(2)),
)

_t2 = pl.pallas_call(
    _t2_body,
    out_shape=jax.ShapeDtypeStruct((2, NR, 128), jnp.float32),
    grid=(GRID,),
    in_specs=[_scr(2), _scr(2), _scr(2), _scr(2), _DS,
              _full((4, 1, 128)), _full((4, 2, 128, 128))],
    out_specs=_scr(2),
)

_t3 = pl.pallas_call(
    _t3_body,
    out_shape=jax.ShapeDtypeStruct((2, NR, 128), jnp.float32),
    grid=(GRID,),
    in_specs=[_scr(2), _scr(2), _DS, _full((2, 1, 128))],
    out_specs=_scr(2),
)

_tf = pl.pallas_call(
    _tf_body,
    out_shape=jax.ShapeDtypeStruct((NR, 8), jnp.float32),
    grid=(GRID,),
    in_specs=[_scr(2), _scr(2), _DS, _full((2, 1, 128)),
              _full((2, 128, 8)), _full((1, 1))],
    out_specs=pl.BlockSpec((BR, 8), lambda i: (i, 0)),
)


_I8 = _np.eye(8, dtype=_np.float32)


def _kron_blocks(w, nq_in, nq_out):
    blk = w.reshape(nq_in, 16, nq_out, 16).transpose(0, 2, 1, 3)
    return jnp.einsum('ij,qpab->qpiajb', _I8, blk).reshape(
        nq_in, nq_out, 128, 128)


def _bias_scr(b, nq):
    return jnp.tile(b.reshape(nq, 1, 16), (1, 1, 8)).reshape(nq, 1, 128)


def _as_sc(v):
    return v.reshape(2, NPAD, W2)


def _as_tc(a):
    return a.reshape(2, NR, 128)


def kernel(x, edge_index, W_fc1, b_fc1, W_c1, b_c1, W_c2, b_c2, W_c3, b_c3,
           W_fc2, b_fc2):
    src = edge_index[0]
    dst = edge_index[1]
    pad = jnp.full((EPAD - E,), N, jnp.int32)
    srcq = jnp.concatenate([src, pad]).reshape(EROWS, 128)
    dstq = jnp.concatenate([dst, pad]).reshape(EROWS, 128)
    xs = jnp.zeros((NPAD, 16), jnp.float32).at[:N, :3].set(x).reshape(NR, 128)

    wf16 = jnp.zeros((16, 32), jnp.float32).at[:3].set(W_fc1)
    k0 = _kron_blocks(wf16, 1, 2)[0]
    k1 = _kron_blocks(W_c1, 2, 4)
    k2 = _kron_blocks(W_c2, 4, 4)
    k3 = _kron_blocks(W_c3, 4, 2)
    kfq = W_fc2.reshape(2, 16, 1)
    kf = jnp.einsum('ij,qab->qiajb', _I8, kfq).reshape(2, 128, 8)
    b0 = _bias_scr(b_fc1, 2)
    b1 = _bias_scr(b_c1, 4)
    b2 = _bias_scr(b_c2, 4)
    b3 = _bias_scr(b_c3, 2)

    deg = _deg_pass(dstq)
    dscr, v = _t0(xs, deg.reshape(2, DR, 128), _SBC, k0, b0)
    ds = dscr.reshape(NR, 128)

    for it in range(4):
        a = _prop16(_as_sc(v), srcq, dstq)
        va, vb = _t1(_as_tc(a), v, ds, k1, b1, k2)
        aa = _prop16(_as_sc(va), srcq, dstq)
        ab = _prop16(_as_sc(vb), srcq, dstq)
        v = _t2(_as_tc(aa), _as_tc(ab), va, vb, ds, b2, k3)
        a = _prop16(_as_sc(v), srcq, dstq)
        if it < 3:
            v = _t3(_as_tc(a), v, ds, b3)
    out = _tf(_as_tc(a), v, ds, b3, kf, b_fc2.reshape(1, 1))
    return out.reshape(NPAD, 1)[:N]
